# Initial kernel scaffold; baseline (speedup 1.0000x reference)
import jax
import jax.numpy as jnp
from jax.experimental import pallas as pl


def kernel(sess_emb):
    raise NotImplementedError("write your pallas kernel here")


# speedup vs baseline: 8.3275x; 8.3275x over previous
import jax
import jax.numpy as jnp
from jax.experimental import pallas as pl
from jax.experimental.pallas import tpu as pltpu

B = 4096
H = 128
K = 3
BLOCK = 256


def _block_kernel(emb_blk_ref, emb_all_ref, out_ref):
    xb = emb_blk_ref[...]
    xa = emb_all_ref[...]

    nb = xb / jnp.maximum(jnp.sqrt(jnp.sum(xb * xb, axis=1, keepdims=True)), 1e-12)
    na = xa / jnp.maximum(jnp.sqrt(jnp.sum(xa * xa, axis=1, keepdims=True)), 1e-12)

    sim = jax.lax.dot_general(
        nb.astype(jnp.bfloat16), na.astype(jnp.bfloat16), (((1,), (1,)), ((), ())),
        preferred_element_type=jnp.float32,
    )

    col = jax.lax.broadcasted_iota(jnp.int32, sim.shape, 1)
    s = sim
    vals = []
    hots = []
    for _ in range(K):
        v = jnp.max(s, axis=1, keepdims=True)
        idx = jnp.argmax(s, axis=1, keepdims=True)
        hot = (col == idx)
        vals.append(v)
        hots.append(hot)
        s = jnp.where(hot, -jnp.inf, s)

    ws = [jnp.exp(v - vals[0]) for v in vals]
    denom = ws[0] + ws[1] + ws[2]
    sel = (ws[0] * hots[0].astype(jnp.float32)
           + ws[1] * hots[1].astype(jnp.float32)
           + ws[2] * hots[2].astype(jnp.float32)) / denom

    out_ref[...] = jax.lax.dot_general(
        sel, xa, (((1,), (0,)), ((), ())),
        preferred_element_type=jnp.float32,
        precision=jax.lax.Precision.HIGHEST,
    )


def kernel(sess_emb):
    grid = (B // BLOCK,)
    return pl.pallas_call(
        _block_kernel,
        grid=grid,
        in_specs=[
            pl.BlockSpec((BLOCK, H), lambda i: (i, 0)),
            pl.BlockSpec((B, H), lambda i: (0, 0)),
        ],
        out_specs=pl.BlockSpec((BLOCK, H), lambda i: (i, 0)),
        out_shape=jax.ShapeDtypeStruct((B, H), jnp.float32),
    )(sess_emb, sess_emb)


# speedup vs baseline: 12.5519x; 1.5073x over previous
import jax
import jax.numpy as jnp
from jax.experimental import pallas as pl
from jax.experimental.pallas import tpu as pltpu

B = 4096
H = 128
BLOCK = 256


def _block_kernel(emb_blk_ref, emb_all_ref, out_ref):
    xb = emb_blk_ref[...]
    xa = emb_all_ref[...]

    nb = xb / jnp.maximum(jnp.sqrt(jnp.sum(xb * xb, axis=1, keepdims=True)), 1e-12)
    na = xa / jnp.maximum(jnp.sqrt(jnp.sum(xa * xa, axis=1, keepdims=True)), 1e-12)

    sim = jax.lax.dot_general(
        nb.astype(jnp.bfloat16), na.astype(jnp.bfloat16), (((1,), (1,)), ((), ())),
        preferred_element_type=jnp.float32,
    )

    col = jax.lax.broadcasted_iota(jnp.int32, sim.shape, 1)
    row = jax.lax.broadcasted_iota(jnp.int32, sim.shape, 0) + pl.program_id(0) * BLOCK

    hot1 = col == row
    v1 = jnp.sum(jnp.where(hot1, sim, 0.0), axis=1, keepdims=True)
    s2 = jnp.where(hot1, -jnp.inf, sim)

    v2 = jnp.max(s2, axis=1, keepdims=True)
    hot2 = col == jnp.argmax(s2, axis=1, keepdims=True)
    s3 = jnp.where(hot2, -jnp.inf, s2)

    v3 = jnp.max(s3, axis=1, keepdims=True)
    hot3 = col == jnp.argmax(s3, axis=1, keepdims=True)

    w2 = jnp.exp(v2 - v1)
    w3 = jnp.exp(v3 - v1)
    denom = 1.0 + w2 + w3

    zero = jnp.zeros_like(sim)
    sel = jnp.where(hot1, 1.0, jnp.where(hot2, w2, jnp.where(hot3, w3, zero)))
    sel16 = sel.astype(jnp.bfloat16)

    xa_hi = xa.astype(jnp.bfloat16)
    xa_lo = (xa - xa_hi.astype(jnp.float32)).astype(jnp.bfloat16)
    dims = (((1,), (0,)), ((), ()))
    agg = (jax.lax.dot_general(sel16, xa_hi, dims, preferred_element_type=jnp.float32)
           + jax.lax.dot_general(sel16, xa_lo, dims, preferred_element_type=jnp.float32))
    out_ref[...] = agg / denom


def kernel(sess_emb):
    grid = (B // BLOCK,)
    return pl.pallas_call(
        _block_kernel,
        grid=grid,
        in_specs=[
            pl.BlockSpec((BLOCK, H), lambda i: (i, 0)),
            pl.BlockSpec((B, H), lambda i: (0, 0)),
        ],
        out_specs=pl.BlockSpec((BLOCK, H), lambda i: (i, 0)),
        out_shape=jax.ShapeDtypeStruct((B, H), jnp.float32),
    )(sess_emb, sess_emb)


# speedup vs baseline: 15.7829x; 1.2574x over previous
import jax
import jax.numpy as jnp
from jax.experimental import pallas as pl
from jax.experimental.pallas import tpu as pltpu

B = 4096
H = 128
BLOCK = 256


def _prep_kernel(emb_ref, n16_ref, hi_ref, lo_ref):
    xa = emb_ref[...]
    na = xa / jnp.maximum(jnp.sqrt(jnp.sum(xa * xa, axis=1, keepdims=True)), 1e-12)
    n16_ref[...] = na.astype(jnp.bfloat16)
    hi = xa.astype(jnp.bfloat16)
    hi_ref[...] = hi
    lo_ref[...] = (xa - hi.astype(jnp.float32)).astype(jnp.bfloat16)


def _block_kernel(nblk_ref, nall_ref, hi_ref, lo_ref, out_ref):
    nb16 = nblk_ref[...]
    na16 = nall_ref[...]

    sim = jax.lax.dot_general(
        nb16, na16, (((1,), (1,)), ((), ())),
        preferred_element_type=jnp.float32,
    )

    col = jax.lax.broadcasted_iota(jnp.int32, sim.shape, 1)
    row = jax.lax.broadcasted_iota(jnp.int32, sim.shape, 0) + pl.program_id(0) * BLOCK

    hot1 = col == row
    nbf = nb16.astype(jnp.float32)
    v1 = jnp.sum(nbf * nbf, axis=1, keepdims=True)
    s2 = jnp.where(hot1, -jnp.inf, sim)

    v2 = jnp.max(s2, axis=1, keepdims=True)
    hot2 = col == jnp.argmax(s2, axis=1, keepdims=True)
    s3 = jnp.where(hot2, -jnp.inf, s2)

    v3 = jnp.max(s3, axis=1, keepdims=True)
    hot3 = col == jnp.argmax(s3, axis=1, keepdims=True)

    w2 = jnp.exp(v2 - v1)
    w3 = jnp.exp(v3 - v1)
    rden = 1.0 / (1.0 + w2 + w3)

    zero = jnp.zeros_like(sim)
    sel = jnp.where(hot1, 1.0, jnp.where(hot2, w2, jnp.where(hot3, w3, zero)))
    sel16 = sel.astype(jnp.bfloat16)

    dims = (((1,), (0,)), ((), ()))
    agg = (jax.lax.dot_general(sel16, hi_ref[...], dims, preferred_element_type=jnp.float32)
           + jax.lax.dot_general(sel16, lo_ref[...], dims, preferred_element_type=jnp.float32))
    out_ref[...] = agg * rden


def kernel(sess_emb):
    n16, hi, lo = pl.pallas_call(
        _prep_kernel,
        out_shape=[
            jax.ShapeDtypeStruct((B, H), jnp.bfloat16),
            jax.ShapeDtypeStruct((B, H), jnp.bfloat16),
            jax.ShapeDtypeStruct((B, H), jnp.bfloat16),
        ],
    )(sess_emb)

    return pl.pallas_call(
        _block_kernel,
        grid=(B // BLOCK,),
        in_specs=[
            pl.BlockSpec((BLOCK, H), lambda i: (i, 0)),
            pl.BlockSpec((B, H), lambda i: (0, 0)),
            pl.BlockSpec((B, H), lambda i: (0, 0)),
            pl.BlockSpec((B, H), lambda i: (0, 0)),
        ],
        out_specs=pl.BlockSpec((BLOCK, H), lambda i: (i, 0)),
        out_shape=jax.ShapeDtypeStruct((B, H), jnp.float32),
    )(n16, n16, hi, lo)


# speedup vs baseline: 15.8035x; 1.0013x over previous
import jax
import jax.numpy as jnp
from jax.experimental import pallas as pl
from jax.experimental.pallas import tpu as pltpu

B = 4096
H = 128
BLOCK = 256


def _prep_kernel(emb_ref, n16_ref, hi_ref, lo_ref):
    xa = emb_ref[...]
    na = xa / jnp.maximum(jnp.sqrt(jnp.sum(xa * xa, axis=1, keepdims=True)), 1e-12)
    n16_ref[...] = na.astype(jnp.bfloat16)
    hi = xa.astype(jnp.bfloat16)
    hi_ref[...] = hi
    lo_ref[...] = (xa - hi.astype(jnp.float32)).astype(jnp.bfloat16)


def _block_kernel(nblk_ref, nall_ref, hi_ref, lo_ref, out_ref):
    nb16 = nblk_ref[...]
    na16 = nall_ref[...]

    sim = jax.lax.dot_general(
        nb16, na16, (((1,), (1,)), ((), ())),
        preferred_element_type=jnp.float32,
    )

    col = jax.lax.broadcasted_iota(jnp.int32, sim.shape, 1)
    row = (jax.lax.broadcasted_iota(jnp.int32, sim.shape, 0)
           + pl.program_id(0) * BLOCK)
    big = jnp.int32(2**30)

    hot1 = col == row
    nbf = nb16.astype(jnp.float32)
    v1 = jnp.sum(nbf * nbf, axis=1, keepdims=True)
    s2 = jnp.where(hot1, -jnp.inf, sim)

    v2 = jnp.max(s2, axis=1, keepdims=True)
    hot2 = col == jnp.argmax(s2, axis=1, keepdims=True)
    s3 = jnp.where(hot2, -jnp.inf, s2)

    v3 = jnp.max(s3, axis=1, keepdims=True)
    hot3 = col == jnp.argmax(s3, axis=1, keepdims=True)

    w2 = jnp.exp(v2 - v1)
    w3 = jnp.exp(v3 - v1)
    rden = 1.0 / (1.0 + w2 + w3)

    zero = jnp.zeros_like(sim)
    sel = jnp.where(hot1, 1.0, jnp.where(hot2, w2, jnp.where(hot3, w3, zero)))
    sel16 = sel.astype(jnp.bfloat16)

    dims = (((1,), (0,)), ((), ()))
    agg = (jax.lax.dot_general(sel16, hi_ref[...], dims, preferred_element_type=jnp.float32)
           + jax.lax.dot_general(sel16, lo_ref[...], dims, preferred_element_type=jnp.float32))
    out_ref[...] = agg * rden


def kernel(sess_emb):
    n16, hi, lo = pl.pallas_call(
        _prep_kernel,
        out_shape=[
            jax.ShapeDtypeStruct((B, H), jnp.bfloat16),
            jax.ShapeDtypeStruct((B, H), jnp.bfloat16),
            jax.ShapeDtypeStruct((B, H), jnp.bfloat16),
        ],
    )(sess_emb)

    return pl.pallas_call(
        _block_kernel,
        grid=(B // BLOCK,),
        in_specs=[
            pl.BlockSpec((BLOCK, H), lambda i: (i, 0)),
            pl.BlockSpec((B, H), lambda i: (0, 0)),
            pl.BlockSpec((B, H), lambda i: (0, 0)),
            pl.BlockSpec((B, H), lambda i: (0, 0)),
        ],
        out_specs=pl.BlockSpec((BLOCK, H), lambda i: (i, 0)),
        out_shape=jax.ShapeDtypeStruct((B, H), jnp.float32),
    )(n16, n16, hi, lo)


# speedup vs baseline: 16.6040x; 1.0507x over previous
import jax
import jax.numpy as jnp
from jax.experimental import pallas as pl
from jax.experimental.pallas import tpu as pltpu

B = 4096
H = 128
BLOCK = 256


def _block_kernel(emb_all_ref, out_ref, n16_s, hi_s, lo_s):
    i = pl.program_id(0)

    @pl.when(i == 0)
    def _prep():
        xa = emb_all_ref[...]
        na = xa / jnp.maximum(jnp.sqrt(jnp.sum(xa * xa, axis=1, keepdims=True)), 1e-12)
        n16_s[...] = na.astype(jnp.bfloat16)
        hi = xa.astype(jnp.bfloat16)
        hi_s[...] = hi
        lo_s[...] = (xa - hi.astype(jnp.float32)).astype(jnp.bfloat16)

    nb16 = n16_s[pl.ds(i * BLOCK, BLOCK), :]
    na16 = n16_s[...]

    sim = jax.lax.dot_general(
        nb16, na16, (((1,), (1,)), ((), ())),
        preferred_element_type=jnp.float32,
    )

    col = jax.lax.broadcasted_iota(jnp.int32, sim.shape, 1)
    row = (jax.lax.broadcasted_iota(jnp.int32, sim.shape, 0) + i * BLOCK)

    hot1 = col == row
    nbf = nb16.astype(jnp.float32)
    v1 = jnp.sum(nbf * nbf, axis=1, keepdims=True)
    s2 = jnp.where(hot1, -jnp.inf, sim)

    v2 = jnp.max(s2, axis=1, keepdims=True)
    hot2 = col == jnp.argmax(s2, axis=1, keepdims=True)
    s3 = jnp.where(hot2, -jnp.inf, s2)

    v3 = jnp.max(s3, axis=1, keepdims=True)
    hot3 = col == jnp.argmax(s3, axis=1, keepdims=True)

    w2 = jnp.exp(v2 - v1)
    w3 = jnp.exp(v3 - v1)
    rden = 1.0 / (1.0 + w2 + w3)

    zero = jnp.zeros_like(sim)
    sel = jnp.where(hot1, 1.0, jnp.where(hot2, w2, jnp.where(hot3, w3, zero)))
    sel16 = sel.astype(jnp.bfloat16)

    dims = (((1,), (0,)), ((), ()))
    agg = (jax.lax.dot_general(sel16, hi_s[...], dims, preferred_element_type=jnp.float32)
           + jax.lax.dot_general(sel16, lo_s[...], dims, preferred_element_type=jnp.float32))
    out_ref[...] = agg * rden


def kernel(sess_emb):
    return pl.pallas_call(
        _block_kernel,
        grid=(B // BLOCK,),
        in_specs=[pl.BlockSpec((B, H), lambda i: (0, 0))],
        out_specs=pl.BlockSpec((BLOCK, H), lambda i: (i, 0)),
        out_shape=jax.ShapeDtypeStruct((B, H), jnp.float32),
        scratch_shapes=[
            pltpu.VMEM((B, H), jnp.bfloat16),
            pltpu.VMEM((B, H), jnp.bfloat16),
            pltpu.VMEM((B, H), jnp.bfloat16),
        ],
    )(sess_emb)


# speedup vs baseline: 17.8346x; 1.0741x over previous
import jax
import jax.numpy as jnp
from jax.experimental import pallas as pl
from jax.experimental.pallas import tpu as pltpu

B = 4096
H = 128
BLOCK = 256


def _block_kernel(emb_all_ref, out_ref, n16_s, hi_s, lo_s):
    i = pl.program_id(0)

    @pl.when(i == 0)
    def _prep():
        xa = emb_all_ref[...]
        na = xa / jnp.maximum(jnp.sqrt(jnp.sum(xa * xa, axis=1, keepdims=True)), 1e-12)
        n16_s[...] = na.astype(jnp.bfloat16)
        hi = xa.astype(jnp.bfloat16)
        hi_s[...] = hi
        lo_s[...] = (xa - hi.astype(jnp.float32)).astype(jnp.bfloat16)

    nb16 = n16_s[pl.ds(i * BLOCK, BLOCK), :]
    na16 = n16_s[...]

    sim = jax.lax.dot_general(
        nb16, na16, (((1,), (1,)), ((), ())),
        preferred_element_type=jnp.float32,
    )

    col = jax.lax.broadcasted_iota(jnp.int32, sim.shape, 1)
    row = (jax.lax.broadcasted_iota(jnp.int32, sim.shape, 0) + i * BLOCK)

    hot1 = col == row
    nbf = nb16.astype(jnp.float32)
    v1 = jnp.sum(nbf * nbf, axis=1, keepdims=True)
    s2 = jnp.where(hot1, -jnp.inf, sim)

    v2 = jnp.max(s2, axis=1, keepdims=True)
    hot2 = col == jnp.argmax(s2, axis=1, keepdims=True)
    s3 = jnp.where(hot2, -jnp.inf, s2)

    v3 = jnp.max(s3, axis=1, keepdims=True)
    hot3 = col == jnp.argmax(s3, axis=1, keepdims=True)

    w2 = jnp.exp(v2 - v1)
    w3 = jnp.exp(v3 - v1)
    rden = 1.0 / (1.0 + w2 + w3)

    zero = jnp.zeros_like(sim)
    sel = jnp.where(hot2, w2, jnp.where(hot3, w3, zero))
    sel16 = sel.astype(jnp.bfloat16)

    xb = emb_all_ref[pl.ds(i * BLOCK, BLOCK), :]
    dims = (((1,), (0,)), ((), ()))
    agg = (jax.lax.dot_general(sel16, hi_s[...], dims, preferred_element_type=jnp.float32)
           + jax.lax.dot_general(sel16, lo_s[...], dims, preferred_element_type=jnp.float32))
    out_ref[...] = (xb + agg) * rden


def kernel(sess_emb):
    return pl.pallas_call(
        _block_kernel,
        grid=(B // BLOCK,),
        in_specs=[pl.BlockSpec((B, H), lambda i: (0, 0))],
        out_specs=pl.BlockSpec((BLOCK, H), lambda i: (i, 0)),
        out_shape=jax.ShapeDtypeStruct((B, H), jnp.float32),
        scratch_shapes=[
            pltpu.VMEM((B, H), jnp.bfloat16),
            pltpu.VMEM((B, H), jnp.bfloat16),
            pltpu.VMEM((B, H), jnp.bfloat16),
        ],
    )(sess_emb)
